# baseline (speedup 1.0000x reference)
import functools

import jax
import jax.numpy as jnp
from jax import lax
from jax.experimental import pallas as pl
from jax.experimental.pallas import tpu as pltpu
from jax.experimental.pallas import tpu_sc as plsc

D = 128
EPS = 1e-5
ALPHA = 0.5


_HI_MASK = -65536


def _bf16_bits(x):
    xi = lax.bitcast_convert_type(x, jnp.int32)
    return (xi + 0x7FFF + ((xi >> 16) & 1)) & _HI_MASK


def _pack2(a, b):
    return _bf16_bits(a) | ((_bf16_bits(b) >> 16) & 0xFFFF)


def _unpack_hi(w):
    return lax.bitcast_convert_type(w & _HI_MASK, jnp.float32)


def _unpack_lo(w):
    return lax.bitcast_convert_type(w << 16, jnp.float32)


def _k0_body(h_ref, wq_ref, q_ref):
    q = jnp.dot(h_ref[...], wq_ref[...], preferred_element_type=jnp.float32)
    q_ref[...] = _pack2(q[:, :D], q[:, D:])


def _k0(h, wq):
    n = h.shape[0]
    return pl.pallas_call(
        _k0_body,
        out_shape=jax.ShapeDtypeStruct((n, D), jnp.int32),
    )(h, wq)


def _sc_gather(q, src, dst):
    e = src.shape[0]
    info = plsc.get_sparse_core_info()
    nc, ns = info.num_cores, info.num_subcores
    nw = nc * ns
    epw = e // nw
    chunk = 200
    nch = epw // chunk
    mesh = plsc.VectorSubcoreMesh(core_axis_name="c", subcore_axis_name="s")

    @functools.partial(
        pl.kernel,
        mesh=mesh,
        out_type=[jax.ShapeDtypeStruct((e, D), jnp.int32),
                  jax.ShapeDtypeStruct((e, D), jnp.int32)],
        scratch_types=[
            pltpu.VMEM((chunk,), jnp.int32),
            pltpu.VMEM((chunk,), jnp.int32),
            pltpu.VMEM((chunk,), jnp.int32),
            pltpu.VMEM((chunk,), jnp.int32),
            pltpu.VMEM((chunk, D), jnp.int32),
            pltpu.VMEM((chunk, D), jnp.int32),
            pltpu.VMEM((chunk, D), jnp.int32),
            pltpu.VMEM((chunk, D), jnp.int32),
            pltpu.SemaphoreType.DMA,
            pltpu.SemaphoreType.DMA,
            pltpu.SemaphoreType.DMA,
            pltpu.SemaphoreType.DMA,
        ],
    )
    def k(q_hbm, src_hbm, dst_hbm, g1_hbm, g2_hbm,
          i1a, i2a, i1b, i2b, r1a, r2a, r1b, r2b,
          s1a, s2a, s1b, s2b):
        wid = lax.axis_index("s") * nc + lax.axis_index("c")
        base = wid * epw

        def load_gather(c, i1, i2, r1, r2, s1, s2):
            o = base + c * chunk
            pltpu.sync_copy(src_hbm.at[pl.ds(o, chunk)], i1)
            pltpu.sync_copy(dst_hbm.at[pl.ds(o, chunk)], i2)
            pltpu.async_copy(q_hbm.at[i1], r1, s1)
            pltpu.async_copy(q_hbm.at[i2], r2, s2)

        def wait_g(r1, r2, s1, s2):
            pltpu.make_async_copy(g1_hbm.at[pl.ds(0, chunk)], r1, s1).wait()
            pltpu.make_async_copy(g1_hbm.at[pl.ds(0, chunk)], r2, s2).wait()

        def write(c, r1, r2):
            o = base + c * chunk
            pltpu.sync_copy(r1, g1_hbm.at[pl.ds(o, chunk)])
            pltpu.sync_copy(r2, g2_hbm.at[pl.ds(o, chunk)])

        load_gather(0, i1a, i2a, r1a, r2a, s1a, s2a)

        def body(j, carry):
            ca = 2 * j
            cb = ca + 1
            wait_g(r1a, r2a, s1a, s2a)
            load_gather(cb, i1b, i2b, r1b, r2b, s1b, s2b)
            write(ca, r1a, r2a)
            wait_g(r1b, r2b, s1b, s2b)

            @pl.when(cb + 1 < nch)
            def _():
                load_gather(cb + 1, i1a, i2a, r1a, r2a, s1a, s2a)

            write(cb, r1b, r2b)
            return carry

        lax.fori_loop(0, nch // 2, body, 0)

    return k(q, src, dst)


def _k2_body(g1_ref, g2_ref, ear_ref, wc_ref, bm1_ref,
             y1p_ref, sst_ref, sts_ref, *, be):
    i = pl.program_id(0)
    ew = jnp.dot(ear_ref[...], wc_ref[...],
                 preferred_element_type=jnp.float32) + bm1_ref[...]
    g1 = g1_ref[...]
    g2 = g2_ref[...]
    yst = _unpack_hi(g1) + _unpack_lo(g2) + ew
    yts = _unpack_hi(g2) + _unpack_lo(g1) + ew
    y1p_ref[...] = _pack2(yst, yts)
    upd_st = jnp.concatenate(
        [jnp.sum(yst, 0, keepdims=True), jnp.sum(yst * yst, 0, keepdims=True)], 0)
    upd_ts = jnp.concatenate(
        [jnp.sum(yts, 0, keepdims=True), jnp.sum(yts * yts, 0, keepdims=True)], 0)

    @pl.when(i == 0)
    def _():
        sst_ref[...] = jnp.zeros_like(sst_ref)
        sts_ref[...] = jnp.zeros_like(sts_ref)

    sst_ref[...] += upd_st
    sts_ref[...] += upd_ts


def _k2(g1, g2, ear, wc, bm1, be):
    e = g1.shape[0]
    nb = e // be
    return pl.pallas_call(
        functools.partial(_k2_body, be=be),
        grid=(nb,),
        in_specs=[
            pl.BlockSpec((be, D), lambda i: (i, 0)),
            pl.BlockSpec((be, D), lambda i: (i, 0)),
            pl.BlockSpec((be, 4), lambda i: (i, 0)),
            pl.BlockSpec((4, D), lambda i: (0, 0)),
            pl.BlockSpec((1, D), lambda i: (0, 0)),
        ],
        out_specs=[
            pl.BlockSpec((be, D), lambda i: (i, 0)),
            pl.BlockSpec((2, D), lambda i: (0, 0)),
            pl.BlockSpec((2, D), lambda i: (0, 0)),
        ],
        out_shape=[
            jax.ShapeDtypeStruct((e, D), jnp.int32),
            jax.ShapeDtypeStruct((2, D), jnp.float32),
            jax.ShapeDtypeStruct((2, D), jnp.float32),
        ],
    )(g1, g2, ear, wc, bm1)


def _scale_shift(stats, g, b, count):
    m = stats[0:1, :] / count
    v = stats[1:2, :] / count - m * m
    scale = g * lax.rsqrt(v + EPS)
    shift = b - m * scale
    return scale, shift


def _k3_body(y1p_ref, s1st_ref, s1ts_ref, gm1_ref, bem1_ref,
             wm2_ref, bm2_ref, y2p_ref, sst_ref, sts_ref, *, count):
    i = pl.program_id(0)
    g = gm1_ref[...]
    b = bem1_ref[...]
    sc_st, sh_st = _scale_shift(s1st_ref[...], g, b, count)
    sc_ts, sh_ts = _scale_shift(s1ts_ref[...], g, b, count)
    y1p = y1p_ref[...]
    z_st = jnp.maximum(_unpack_hi(y1p) * sc_st + sh_st, 0.0)
    z_ts = jnp.maximum(_unpack_lo(y1p) * sc_ts + sh_ts, 0.0)
    w = wm2_ref[...].astype(jnp.bfloat16)
    y2st = jnp.dot(z_st.astype(jnp.bfloat16), w,
                   preferred_element_type=jnp.float32) + bm2_ref[...]
    y2ts = jnp.dot(z_ts.astype(jnp.bfloat16), w,
                   preferred_element_type=jnp.float32) + bm2_ref[...]
    y2p_ref[...] = _pack2(y2st, y2ts)
    upd_st = jnp.concatenate(
        [jnp.sum(y2st, 0, keepdims=True), jnp.sum(y2st * y2st, 0, keepdims=True)], 0)
    upd_ts = jnp.concatenate(
        [jnp.sum(y2ts, 0, keepdims=True), jnp.sum(y2ts * y2ts, 0, keepdims=True)], 0)

    @pl.when(i == 0)
    def _():
        sst_ref[...] = jnp.zeros_like(sst_ref)
        sts_ref[...] = jnp.zeros_like(sts_ref)

    sst_ref[...] += upd_st
    sts_ref[...] += upd_ts


def _k3(y1p, s1st, s1ts, gm1, bem1, wm2, bm2, be):
    e = y1p.shape[0]
    nb = e // be
    blk = lambda i: (i, 0)
    fix = lambda i: (0, 0)
    return pl.pallas_call(
        functools.partial(_k3_body, count=float(e)),
        grid=(nb,),
        in_specs=[
            pl.BlockSpec((be, D), blk),
            pl.BlockSpec((2, D), fix),
            pl.BlockSpec((2, D), fix),
            pl.BlockSpec((1, D), fix),
            pl.BlockSpec((1, D), fix),
            pl.BlockSpec((D, D), fix),
            pl.BlockSpec((1, D), fix),
        ],
        out_specs=[
            pl.BlockSpec((be, D), blk),
            pl.BlockSpec((2, D), fix),
            pl.BlockSpec((2, D), fix),
        ],
        out_shape=[
            jax.ShapeDtypeStruct((e, D), jnp.int32),
            jax.ShapeDtypeStruct((2, D), jnp.float32),
            jax.ShapeDtypeStruct((2, D), jnp.float32),
        ],
    )(y1p, s1st, s1ts, gm1, bem1, wm2, bm2)


def _k3b_body(y2p_ref, s2st_ref, s2ts_ref, gm2_ref, bem2_ref,
              mst_ref, mts_ref, *, count):
    g = gm2_ref[...]
    b = bem2_ref[...]
    sc_st, sh_st = _scale_shift(s2st_ref[...], g, b, count)
    sc_ts, sh_ts = _scale_shift(s2ts_ref[...], g, b, count)
    y2p = y2p_ref[...]
    mst_ref[...] = jnp.maximum(_unpack_hi(y2p) * sc_st + sh_st, 0.0)
    mts_ref[...] = jnp.maximum(_unpack_lo(y2p) * sc_ts + sh_ts, 0.0)


def _k3b(y2p, s2st, s2ts, gm2, bem2, be):
    e = y2p.shape[0]
    nb = e // be
    blk = lambda i: (i, 0)
    fix = lambda i: (0, 0)
    return pl.pallas_call(
        functools.partial(_k3b_body, count=float(e)),
        grid=(nb,),
        in_specs=[
            pl.BlockSpec((be, D), blk),
            pl.BlockSpec((2, D), fix),
            pl.BlockSpec((2, D), fix),
            pl.BlockSpec((1, D), fix),
            pl.BlockSpec((1, D), fix),
        ],
        out_specs=[
            pl.BlockSpec((be, D), blk),
            pl.BlockSpec((be, D), blk),
        ],
        out_shape=[
            jax.ShapeDtypeStruct((e, D), jnp.float32),
            jax.ShapeDtypeStruct((e, D), jnp.float32),
        ],
    )(y2p, s2st, s2ts, gm2, bem2)


def _sc_scatter(mst, mts, src, zeros_n):
    e = src.shape[0]
    n = zeros_n.shape[0]
    info = plsc.get_sparse_core_info()
    nc, ns = info.num_cores, info.num_subcores
    ept = e // ns
    chunk = 80
    nch = ept // chunk
    npt = (n // ns) // 8 * 8
    tail = n - ns * npt
    mesh = plsc.VectorSubcoreMesh(core_axis_name="c", subcore_axis_name="s")

    @functools.partial(
        pl.kernel,
        mesh=mesh,
        out_type=[jax.ShapeDtypeStruct((n, D), jnp.float32),
                  jax.ShapeDtypeStruct((n, D), jnp.float32)],
        scratch_types=[
            pltpu.VMEM((chunk,), jnp.int32),
            pltpu.VMEM((chunk,), jnp.int32),
            pltpu.VMEM((chunk, D), jnp.float32),
            pltpu.VMEM((chunk, D), jnp.float32),
            pltpu.VMEM_SHARED((n, D), jnp.float32),
            pltpu.SemaphoreType.DMA,
            pltpu.SemaphoreType.DMA,
        ],
    )
    def k(mst_hbm, mts_hbm, src_hbm, zeros_hbm, ast_hbm, ats_hbm,
          ia, ib, va, vb, aggr_sh, sa, sb):
        c = lax.axis_index("c")
        s = lax.axis_index("s")
        rs = s * npt
        pltpu.sync_copy(zeros_hbm.at[pl.ds(rs, npt)],
                        aggr_sh.at[pl.ds(rs, npt)])

        @pl.when(s == 0)
        def _():
            pltpu.sync_copy(zeros_hbm.at[pl.ds(ns * npt, tail)],
                            aggr_sh.at[pl.ds(ns * npt, tail)])

        plsc.subcore_barrier()

        base = s * ept

        def load(ci, iv, vv, sem):
            o = base + ci * chunk
            pltpu.sync_copy(src_hbm.at[pl.ds(o, chunk)], iv)

            @pl.when(c == 0)
            def _():
                pltpu.async_copy(mst_hbm.at[pl.ds(o, chunk)], vv, sem)

            @pl.when(c != 0)
            def _():
                pltpu.async_copy(mts_hbm.at[pl.ds(o, chunk)], vv, sem)

        def wait_v(vv, sem):
            pltpu.make_async_copy(mst_hbm.at[pl.ds(0, chunk)], vv, sem).wait()

        load(0, ia, va, sa)

        def body(j, carry):
            ca = 2 * j
            cb = ca + 1
            wait_v(va, sa)
            load(cb, ib, vb, sb)
            pltpu.sync_copy(va, aggr_sh.at[ia], add=True)
            wait_v(vb, sb)

            @pl.when(cb + 1 < nch)
            def _():
                load(cb + 1, ia, va, sa)

            pltpu.sync_copy(vb, aggr_sh.at[ib], add=True)
            return carry

        lax.fori_loop(0, nch // 2, body, 0)
        plsc.subcore_barrier()

        @pl.when(c == 0)
        def _():
            pltpu.sync_copy(aggr_sh.at[pl.ds(rs, npt)],
                            ast_hbm.at[pl.ds(rs, npt)])

            @pl.when(s == 0)
            def _():
                pltpu.sync_copy(aggr_sh.at[pl.ds(ns * npt, tail)],
                                ast_hbm.at[pl.ds(ns * npt, tail)])

        @pl.when(c != 0)
        def _():
            pltpu.sync_copy(aggr_sh.at[pl.ds(rs, npt)],
                            ats_hbm.at[pl.ds(rs, npt)])

            @pl.when(s == 0)
            def _():
                pltpu.sync_copy(aggr_sh.at[pl.ds(ns * npt, tail)],
                                ats_hbm.at[pl.ds(ns * npt, tail)])

    return k(mst, mts, src, zeros_n)


def _k5_body(h_ref, ast_ref, ats_ref, wu1_ref, bu1_ref, gu1_ref, beu1_ref,
             wu2_ref, bu2_ref, gu2_ref, beu2_ref, out_ref):
    h = h_ref[...]
    wu1 = wu1_ref[...]
    u = (jnp.dot(h, wu1[:D], preferred_element_type=jnp.float32)
         + jnp.dot(ast_ref[...], ALPHA * wu1[D:2 * D],
                   preferred_element_type=jnp.float32)
         + jnp.dot(ats_ref[...], (1.0 - ALPHA) * wu1[2 * D:],
                   preferred_element_type=jnp.float32)
         + bu1_ref[...])
    m = jnp.mean(u, axis=0, keepdims=True)
    v = jnp.mean((u - m) * (u - m), axis=0, keepdims=True)
    z = jnp.maximum(gu1_ref[...] * (u - m) * lax.rsqrt(v + EPS)
                    + beu1_ref[...], 0.0)
    u2 = jnp.dot(z, wu2_ref[...], preferred_element_type=jnp.float32) \
        + bu2_ref[...]
    m2 = jnp.mean(u2, axis=0, keepdims=True)
    v2 = jnp.mean((u2 - m2) * (u2 - m2), axis=0, keepdims=True)
    out_ref[...] = jnp.maximum(
        gu2_ref[...] * (u2 - m2) * lax.rsqrt(v2 + EPS) + beu2_ref[...], 0.0)


def _k5(h, ast, ats, wu1, bu1, gu1, beu1, wu2, bu2, gu2, beu2):
    n = h.shape[0]
    return pl.pallas_call(
        _k5_body,
        out_shape=jax.ShapeDtypeStruct((n, D), jnp.float32),
    )(h, ast, ats, wu1, bu1, gu1, beu1, wu2, bu2, gu2, beu2)


_BE = 2560


def kernel(h, edge_index, edge_attr, Wm1, bm1, gm1, bem1, Wm2, bm2, gm2, bem2,
           Wu1, bu1, gu1, beu1, Wu2, bu2, gu2, beu2):
    n = h.shape[0]
    e = edge_index.shape[1]
    src = edge_index[0]
    dst = edge_index[1]
    wq = jnp.concatenate([Wm1[:D], Wm1[D:2 * D]], axis=1)
    wc = Wm1[2 * D:]
    ear = edge_attr
    row = lambda x: x.reshape(1, D)

    q = _k0(h, wq)
    g1, g2 = _sc_gather(q, src, dst)
    y1p, s1st, s1ts = _k2(g1, g2, ear, wc, row(bm1), _BE)
    y2p, s2st, s2ts = _k3(y1p, s1st, s1ts, row(gm1), row(bem1),
                          Wm2, row(bm2), _BE)
    mst, mts = _k3b(y2p, s2st, s2ts, row(gm2), row(bem2), _BE)
    zeros_n = jnp.zeros((n, D), jnp.float32)
    ast, ats = _sc_scatter(mst, mts, src, zeros_n)
    out = _k5(h, ast, ats, Wu1, row(bu1), row(gu1), row(beu1),
              Wu2, row(bu2), row(gu2), row(beu2))
    return out


# speedup vs baseline: 1.0968x; 1.0968x over previous
import functools

import jax
import jax.numpy as jnp
from jax import lax
from jax.experimental import pallas as pl
from jax.experimental.pallas import tpu as pltpu
from jax.experimental.pallas import tpu_sc as plsc

D = 128
EPS = 1e-5
ALPHA = 0.5


_HI_MASK = -65536


def _bf16_bits(x):
    xi = lax.bitcast_convert_type(x, jnp.int32)
    return (xi + 0x7FFF + ((xi >> 16) & 1)) & _HI_MASK


def _pack2(a, b):
    return _bf16_bits(a) | ((_bf16_bits(b) >> 16) & 0xFFFF)


def _unpack_hi(w):
    return lax.bitcast_convert_type(w & _HI_MASK, jnp.float32)


def _unpack_lo(w):
    return lax.bitcast_convert_type(w << 16, jnp.float32)


def _k0_body(h_ref, wq_ref, q_ref):
    q = jnp.dot(h_ref[...], wq_ref[...], preferred_element_type=jnp.float32)
    q_ref[...] = _pack2(q[:, :D], q[:, D:])


def _k0(h, wq):
    n = h.shape[0]
    return pl.pallas_call(
        _k0_body,
        out_shape=jax.ShapeDtypeStruct((n, D), jnp.int32),
    )(h, wq)


def _sc_gather(q, src, dst):
    e = src.shape[0]
    info = plsc.get_sparse_core_info()
    nc, ns = info.num_cores, info.num_subcores
    nw = nc * ns
    epw = e // nw
    chunk = 200
    nch = epw // chunk
    mesh = plsc.VectorSubcoreMesh(core_axis_name="c", subcore_axis_name="s")

    @functools.partial(
        pl.kernel,
        mesh=mesh,
        out_type=[jax.ShapeDtypeStruct((e, D), jnp.int32),
                  jax.ShapeDtypeStruct((e, D), jnp.int32)],
        scratch_types=[
            pltpu.VMEM((chunk,), jnp.int32),
            pltpu.VMEM((chunk,), jnp.int32),
            pltpu.VMEM((chunk,), jnp.int32),
            pltpu.VMEM((chunk,), jnp.int32),
            pltpu.VMEM((chunk, D), jnp.int32),
            pltpu.VMEM((chunk, D), jnp.int32),
            pltpu.VMEM((chunk, D), jnp.int32),
            pltpu.VMEM((chunk, D), jnp.int32),
            pltpu.SemaphoreType.DMA,
            pltpu.SemaphoreType.DMA,
            pltpu.SemaphoreType.DMA,
            pltpu.SemaphoreType.DMA,
        ],
    )
    def k(q_hbm, src_hbm, dst_hbm, g1_hbm, g2_hbm,
          i1a, i2a, i1b, i2b, r1a, r2a, r1b, r2b,
          s1a, s2a, s1b, s2b):
        wid = lax.axis_index("s") * nc + lax.axis_index("c")
        base = wid * epw

        def load_gather(c, i1, i2, r1, r2, s1, s2):
            o = base + c * chunk
            pltpu.sync_copy(src_hbm.at[pl.ds(o, chunk)], i1)
            pltpu.sync_copy(dst_hbm.at[pl.ds(o, chunk)], i2)
            pltpu.async_copy(q_hbm.at[i1], r1, s1)
            pltpu.async_copy(q_hbm.at[i2], r2, s2)

        def wait_g(r1, r2, s1, s2):
            pltpu.make_async_copy(g1_hbm.at[pl.ds(0, chunk)], r1, s1).wait()
            pltpu.make_async_copy(g1_hbm.at[pl.ds(0, chunk)], r2, s2).wait()

        def write(c, r1, r2):
            o = base + c * chunk
            pltpu.sync_copy(r1, g1_hbm.at[pl.ds(o, chunk)])
            pltpu.sync_copy(r2, g2_hbm.at[pl.ds(o, chunk)])

        load_gather(0, i1a, i2a, r1a, r2a, s1a, s2a)

        def body(j, carry):
            ca = 2 * j
            cb = ca + 1
            wait_g(r1a, r2a, s1a, s2a)
            load_gather(cb, i1b, i2b, r1b, r2b, s1b, s2b)
            write(ca, r1a, r2a)
            wait_g(r1b, r2b, s1b, s2b)

            @pl.when(cb + 1 < nch)
            def _():
                load_gather(cb + 1, i1a, i2a, r1a, r2a, s1a, s2a)

            write(cb, r1b, r2b)
            return carry

        lax.fori_loop(0, nch // 2, body, 0)

    return k(q, src, dst)


def _k2_body(g1_ref, g2_ref, ear_ref, wc_ref, bm1_ref,
             y1p_ref, sst_ref, sts_ref, *, be):
    i = pl.program_id(0)
    ew = jnp.dot(ear_ref[...], wc_ref[...],
                 preferred_element_type=jnp.float32) + bm1_ref[...]
    g1 = g1_ref[...]
    g2 = g2_ref[...]
    yst = _unpack_hi(g1) + _unpack_lo(g2) + ew
    yts = _unpack_hi(g2) + _unpack_lo(g1) + ew
    y1p_ref[...] = _pack2(yst, yts)
    upd_st = jnp.concatenate(
        [jnp.sum(yst, 0, keepdims=True), jnp.sum(yst * yst, 0, keepdims=True)], 0)
    upd_ts = jnp.concatenate(
        [jnp.sum(yts, 0, keepdims=True), jnp.sum(yts * yts, 0, keepdims=True)], 0)

    @pl.when(i == 0)
    def _():
        sst_ref[...] = jnp.zeros_like(sst_ref)
        sts_ref[...] = jnp.zeros_like(sts_ref)

    sst_ref[...] += upd_st
    sts_ref[...] += upd_ts


def _k2(g1, g2, ear, wc, bm1, be):
    e = g1.shape[0]
    nb = e // be
    return pl.pallas_call(
        functools.partial(_k2_body, be=be),
        grid=(nb,),
        in_specs=[
            pl.BlockSpec((be, D), lambda i: (i, 0)),
            pl.BlockSpec((be, D), lambda i: (i, 0)),
            pl.BlockSpec((be, 4), lambda i: (i, 0)),
            pl.BlockSpec((4, D), lambda i: (0, 0)),
            pl.BlockSpec((1, D), lambda i: (0, 0)),
        ],
        out_specs=[
            pl.BlockSpec((be, D), lambda i: (i, 0)),
            pl.BlockSpec((2, D), lambda i: (0, 0)),
            pl.BlockSpec((2, D), lambda i: (0, 0)),
        ],
        out_shape=[
            jax.ShapeDtypeStruct((e, D), jnp.int32),
            jax.ShapeDtypeStruct((2, D), jnp.float32),
            jax.ShapeDtypeStruct((2, D), jnp.float32),
        ],
    )(g1, g2, ear, wc, bm1)


def _scale_shift(stats, g, b, count):
    m = stats[0:1, :] / count
    v = stats[1:2, :] / count - m * m
    scale = g * lax.rsqrt(v + EPS)
    shift = b - m * scale
    return scale, shift


def _k3_body(y1p_ref, s1st_ref, s1ts_ref, gm1_ref, bem1_ref,
             wm2_ref, bm2_ref, y2p_ref, sst_ref, sts_ref, *, count):
    i = pl.program_id(0)
    g = gm1_ref[...]
    b = bem1_ref[...]
    sc_st, sh_st = _scale_shift(s1st_ref[...], g, b, count)
    sc_ts, sh_ts = _scale_shift(s1ts_ref[...], g, b, count)
    y1p = y1p_ref[...]
    z_st = jnp.maximum(_unpack_hi(y1p) * sc_st + sh_st, 0.0)
    z_ts = jnp.maximum(_unpack_lo(y1p) * sc_ts + sh_ts, 0.0)
    w = wm2_ref[...].astype(jnp.bfloat16)
    y2st = jnp.dot(z_st.astype(jnp.bfloat16), w,
                   preferred_element_type=jnp.float32) + bm2_ref[...]
    y2ts = jnp.dot(z_ts.astype(jnp.bfloat16), w,
                   preferred_element_type=jnp.float32) + bm2_ref[...]
    y2p_ref[...] = _pack2(y2st, y2ts)
    upd_st = jnp.concatenate(
        [jnp.sum(y2st, 0, keepdims=True), jnp.sum(y2st * y2st, 0, keepdims=True)], 0)
    upd_ts = jnp.concatenate(
        [jnp.sum(y2ts, 0, keepdims=True), jnp.sum(y2ts * y2ts, 0, keepdims=True)], 0)

    @pl.when(i == 0)
    def _():
        sst_ref[...] = jnp.zeros_like(sst_ref)
        sts_ref[...] = jnp.zeros_like(sts_ref)

    sst_ref[...] += upd_st
    sts_ref[...] += upd_ts


def _k3(y1p, s1st, s1ts, gm1, bem1, wm2, bm2, be):
    e = y1p.shape[0]
    nb = e // be
    blk = lambda i: (i, 0)
    fix = lambda i: (0, 0)
    return pl.pallas_call(
        functools.partial(_k3_body, count=float(e)),
        grid=(nb,),
        in_specs=[
            pl.BlockSpec((be, D), blk),
            pl.BlockSpec((2, D), fix),
            pl.BlockSpec((2, D), fix),
            pl.BlockSpec((1, D), fix),
            pl.BlockSpec((1, D), fix),
            pl.BlockSpec((D, D), fix),
            pl.BlockSpec((1, D), fix),
        ],
        out_specs=[
            pl.BlockSpec((be, D), blk),
            pl.BlockSpec((2, D), fix),
            pl.BlockSpec((2, D), fix),
        ],
        out_shape=[
            jax.ShapeDtypeStruct((e, D), jnp.int32),
            jax.ShapeDtypeStruct((2, D), jnp.float32),
            jax.ShapeDtypeStruct((2, D), jnp.float32),
        ],
    )(y1p, s1st, s1ts, gm1, bem1, wm2, bm2)


def _k3b_body(y2p_ref, s2st_ref, s2ts_ref, gm2_ref, bem2_ref,
              mst_ref, mts_ref, *, count):
    g = gm2_ref[...]
    b = bem2_ref[...]
    sc_st, sh_st = _scale_shift(s2st_ref[...], g, b, count)
    sc_ts, sh_ts = _scale_shift(s2ts_ref[...], g, b, count)
    y2p = y2p_ref[...]
    mst_ref[...] = jnp.maximum(_unpack_hi(y2p) * sc_st + sh_st, 0.0)
    mts_ref[...] = jnp.maximum(_unpack_lo(y2p) * sc_ts + sh_ts, 0.0)


def _k3b(y2p, s2st, s2ts, gm2, bem2, be):
    e = y2p.shape[0]
    nb = e // be
    blk = lambda i: (i, 0)
    fix = lambda i: (0, 0)
    return pl.pallas_call(
        functools.partial(_k3b_body, count=float(e)),
        grid=(nb,),
        in_specs=[
            pl.BlockSpec((be, D), blk),
            pl.BlockSpec((2, D), fix),
            pl.BlockSpec((2, D), fix),
            pl.BlockSpec((1, D), fix),
            pl.BlockSpec((1, D), fix),
        ],
        out_specs=[
            pl.BlockSpec((be, D), blk),
            pl.BlockSpec((be, D), blk),
        ],
        out_shape=[
            jax.ShapeDtypeStruct((e, D), jnp.float32),
            jax.ShapeDtypeStruct((e, D), jnp.float32),
        ],
    )(y2p, s2st, s2ts, gm2, bem2)


def _sc_scatter(mst, mts, src, zeros_n):
    e = src.shape[0]
    n = zeros_n.shape[0]
    info = plsc.get_sparse_core_info()
    nc, ns = info.num_cores, info.num_subcores
    ept = e // ns
    chunk = 160
    nch = ept // chunk
    npt = (n // ns) // 8 * 8
    tail = n - ns * npt
    mesh = plsc.VectorSubcoreMesh(core_axis_name="c", subcore_axis_name="s")

    @functools.partial(
        pl.kernel,
        mesh=mesh,
        out_type=[jax.ShapeDtypeStruct((n, D), jnp.float32),
                  jax.ShapeDtypeStruct((n, D), jnp.float32)],
        scratch_types=[
            pltpu.VMEM((chunk,), jnp.int32),
            pltpu.VMEM((chunk,), jnp.int32),
            pltpu.VMEM((chunk, D), jnp.float32),
            pltpu.VMEM((chunk, D), jnp.float32),
            pltpu.VMEM_SHARED((n, D), jnp.float32),
            pltpu.SemaphoreType.DMA,
            pltpu.SemaphoreType.DMA,
        ],
    )
    def k(mst_hbm, mts_hbm, src_hbm, zeros_hbm, ast_hbm, ats_hbm,
          ia, ib, va, vb, aggr_sh, sa, sb):
        c = lax.axis_index("c")
        s = lax.axis_index("s")
        rs = s * npt
        pltpu.sync_copy(zeros_hbm.at[pl.ds(rs, npt)],
                        aggr_sh.at[pl.ds(rs, npt)])

        @pl.when(s == 0)
        def _():
            pltpu.sync_copy(zeros_hbm.at[pl.ds(ns * npt, tail)],
                            aggr_sh.at[pl.ds(ns * npt, tail)])

        plsc.subcore_barrier()

        base = s * ept

        def load(ci, iv, vv, sem):
            o = base + ci * chunk
            pltpu.sync_copy(src_hbm.at[pl.ds(o, chunk)], iv)

            @pl.when(c == 0)
            def _():
                pltpu.async_copy(mst_hbm.at[pl.ds(o, chunk)], vv, sem)

            @pl.when(c != 0)
            def _():
                pltpu.async_copy(mts_hbm.at[pl.ds(o, chunk)], vv, sem)

        def wait_v(vv, sem):
            pltpu.make_async_copy(mst_hbm.at[pl.ds(0, chunk)], vv, sem).wait()

        load(0, ia, va, sa)

        def body(j, carry):
            ca = 2 * j
            cb = ca + 1
            wait_v(va, sa)
            load(cb, ib, vb, sb)
            pltpu.sync_copy(va, aggr_sh.at[ia], add=True)
            wait_v(vb, sb)

            @pl.when(cb + 1 < nch)
            def _():
                load(cb + 1, ia, va, sa)

            pltpu.sync_copy(vb, aggr_sh.at[ib], add=True)
            return carry

        lax.fori_loop(0, nch // 2, body, 0)

        if nch % 2 == 1:
            wait_v(va, sa)
            pltpu.sync_copy(va, aggr_sh.at[ia], add=True)

        plsc.subcore_barrier()

        @pl.when(c == 0)
        def _():
            pltpu.sync_copy(aggr_sh.at[pl.ds(rs, npt)],
                            ast_hbm.at[pl.ds(rs, npt)])

            @pl.when(s == 0)
            def _():
                pltpu.sync_copy(aggr_sh.at[pl.ds(ns * npt, tail)],
                                ast_hbm.at[pl.ds(ns * npt, tail)])

        @pl.when(c != 0)
        def _():
            pltpu.sync_copy(aggr_sh.at[pl.ds(rs, npt)],
                            ats_hbm.at[pl.ds(rs, npt)])

            @pl.when(s == 0)
            def _():
                pltpu.sync_copy(aggr_sh.at[pl.ds(ns * npt, tail)],
                                ats_hbm.at[pl.ds(ns * npt, tail)])

    return k(mst, mts, src, zeros_n)


def _k5_body(h_ref, ast_ref, ats_ref, wu1_ref, bu1_ref, gu1_ref, beu1_ref,
             wu2_ref, bu2_ref, gu2_ref, beu2_ref, out_ref):
    h = h_ref[...]
    wu1 = wu1_ref[...]
    u = (jnp.dot(h, wu1[:D], preferred_element_type=jnp.float32)
         + jnp.dot(ast_ref[...], ALPHA * wu1[D:2 * D],
                   preferred_element_type=jnp.float32)
         + jnp.dot(ats_ref[...], (1.0 - ALPHA) * wu1[2 * D:],
                   preferred_element_type=jnp.float32)
         + bu1_ref[...])
    m = jnp.mean(u, axis=0, keepdims=True)
    v = jnp.mean((u - m) * (u - m), axis=0, keepdims=True)
    z = jnp.maximum(gu1_ref[...] * (u - m) * lax.rsqrt(v + EPS)
                    + beu1_ref[...], 0.0)
    u2 = jnp.dot(z, wu2_ref[...], preferred_element_type=jnp.float32) \
        + bu2_ref[...]
    m2 = jnp.mean(u2, axis=0, keepdims=True)
    v2 = jnp.mean((u2 - m2) * (u2 - m2), axis=0, keepdims=True)
    out_ref[...] = jnp.maximum(
        gu2_ref[...] * (u2 - m2) * lax.rsqrt(v2 + EPS) + beu2_ref[...], 0.0)


def _k5(h, ast, ats, wu1, bu1, gu1, beu1, wu2, bu2, gu2, beu2):
    n = h.shape[0]
    return pl.pallas_call(
        _k5_body,
        out_shape=jax.ShapeDtypeStruct((n, D), jnp.float32),
    )(h, ast, ats, wu1, bu1, gu1, beu1, wu2, bu2, gu2, beu2)


_BE = 2560


def kernel(h, edge_index, edge_attr, Wm1, bm1, gm1, bem1, Wm2, bm2, gm2, bem2,
           Wu1, bu1, gu1, beu1, Wu2, bu2, gu2, beu2):
    n = h.shape[0]
    e = edge_index.shape[1]
    src = edge_index[0]
    dst = edge_index[1]
    wq = jnp.concatenate([Wm1[:D], Wm1[D:2 * D]], axis=1)
    wc = Wm1[2 * D:]
    ear = edge_attr
    row = lambda x: x.reshape(1, D)

    q = _k0(h, wq)
    g1, g2 = _sc_gather(q, src, dst)
    y1p, s1st, s1ts = _k2(g1, g2, ear, wc, row(bm1), _BE)
    y2p, s2st, s2ts = _k3(y1p, s1st, s1ts, row(gm1), row(bem1),
                          Wm2, row(bm2), _BE)
    mst, mts = _k3b(y2p, s2st, s2ts, row(gm2), row(bem2), _BE)
    zeros_n = jnp.zeros((n, D), jnp.float32)
    ast, ats = _sc_scatter(mst, mts, src, zeros_n)
    out = _k5(h, ast, ats, Wu1, row(bu1), row(gu1), row(beu1),
              Wu2, row(bu2), row(gu2), row(beu2))
    return out


# speedup vs baseline: 1.1889x; 1.0839x over previous
import functools

import jax
import jax.numpy as jnp
from jax import lax
from jax.experimental import pallas as pl
from jax.experimental.pallas import tpu as pltpu
from jax.experimental.pallas import tpu_sc as plsc

D = 128
EPS = 1e-5
ALPHA = 0.5


_HI_MASK = -65536


def _bf16_bits(x):
    xi = lax.bitcast_convert_type(x, jnp.int32)
    return (xi + 0x7FFF + ((xi >> 16) & 1)) & _HI_MASK


def _pack2(a, b):
    return _bf16_bits(a) | ((_bf16_bits(b) >> 16) & 0xFFFF)


def _unpack_hi(w):
    return lax.bitcast_convert_type(w & _HI_MASK, jnp.float32)


def _unpack_lo(w):
    return lax.bitcast_convert_type(w << 16, jnp.float32)


def _k0_body(h_ref, wq_ref, q_ref):
    q = jnp.dot(h_ref[...], wq_ref[...], preferred_element_type=jnp.float32)
    q_ref[...] = _pack2(q[:, :D], q[:, D:])


def _k0(h, wq):
    n = h.shape[0]
    return pl.pallas_call(
        _k0_body,
        out_shape=jax.ShapeDtypeStruct((n, D), jnp.int32),
    )(h, wq)


def _sc_gather(q, src, dst):
    e = src.shape[0]
    info = plsc.get_sparse_core_info()
    nc, ns = info.num_cores, info.num_subcores
    nw = nc * ns
    epw = e // nw
    chunk = 200
    nch = epw // chunk
    mesh = plsc.VectorSubcoreMesh(core_axis_name="c", subcore_axis_name="s")

    @functools.partial(
        pl.kernel,
        mesh=mesh,
        out_type=[jax.ShapeDtypeStruct((e, D), jnp.int32),
                  jax.ShapeDtypeStruct((e, D), jnp.int32)],
        scratch_types=[
            pltpu.VMEM((chunk,), jnp.int32),
            pltpu.VMEM((chunk,), jnp.int32),
            pltpu.VMEM((chunk,), jnp.int32),
            pltpu.VMEM((chunk,), jnp.int32),
            pltpu.VMEM((chunk, D), jnp.int32),
            pltpu.VMEM((chunk, D), jnp.int32),
            pltpu.VMEM((chunk, D), jnp.int32),
            pltpu.VMEM((chunk, D), jnp.int32),
            pltpu.SemaphoreType.DMA,
            pltpu.SemaphoreType.DMA,
            pltpu.SemaphoreType.DMA,
            pltpu.SemaphoreType.DMA,
        ],
    )
    def k(q_hbm, src_hbm, dst_hbm, g1_hbm, g2_hbm,
          i1a, i2a, i1b, i2b, r1a, r2a, r1b, r2b,
          s1a, s2a, s1b, s2b):
        wid = lax.axis_index("s") * nc + lax.axis_index("c")
        base = wid * epw

        def load_gather(c, i1, i2, r1, r2, s1, s2):
            o = base + c * chunk
            pltpu.sync_copy(src_hbm.at[pl.ds(o, chunk)], i1)
            pltpu.sync_copy(dst_hbm.at[pl.ds(o, chunk)], i2)
            pltpu.async_copy(q_hbm.at[i1], r1, s1)
            pltpu.async_copy(q_hbm.at[i2], r2, s2)

        def wait_g(r1, r2, s1, s2):
            pltpu.make_async_copy(g1_hbm.at[pl.ds(0, chunk)], r1, s1).wait()
            pltpu.make_async_copy(g1_hbm.at[pl.ds(0, chunk)], r2, s2).wait()

        def write(c, r1, r2):
            o = base + c * chunk
            pltpu.sync_copy(r1, g1_hbm.at[pl.ds(o, chunk)])
            pltpu.sync_copy(r2, g2_hbm.at[pl.ds(o, chunk)])

        load_gather(0, i1a, i2a, r1a, r2a, s1a, s2a)

        def body(j, carry):
            ca = 2 * j
            cb = ca + 1
            wait_g(r1a, r2a, s1a, s2a)
            load_gather(cb, i1b, i2b, r1b, r2b, s1b, s2b)
            write(ca, r1a, r2a)
            wait_g(r1b, r2b, s1b, s2b)

            @pl.when(cb + 1 < nch)
            def _():
                load_gather(cb + 1, i1a, i2a, r1a, r2a, s1a, s2a)

            write(cb, r1b, r2b)
            return carry

        lax.fori_loop(0, nch // 2, body, 0)

    return k(q, src, dst)


def _k2_body(g1_ref, g2_ref, ear_ref, wc_ref, bm1_ref,
             y1p_ref, sst_ref, sts_ref, *, be):
    i = pl.program_id(0)
    ew = jnp.dot(ear_ref[...], wc_ref[...],
                 preferred_element_type=jnp.float32) + bm1_ref[...]
    g1 = g1_ref[...]
    g2 = g2_ref[...]
    yst = _unpack_hi(g1) + _unpack_lo(g2) + ew
    yts = _unpack_hi(g2) + _unpack_lo(g1) + ew
    y1p_ref[...] = _pack2(yst, yts)
    upd_st = jnp.concatenate(
        [jnp.sum(yst, 0, keepdims=True), jnp.sum(yst * yst, 0, keepdims=True)], 0)
    upd_ts = jnp.concatenate(
        [jnp.sum(yts, 0, keepdims=True), jnp.sum(yts * yts, 0, keepdims=True)], 0)

    @pl.when(i == 0)
    def _():
        sst_ref[...] = jnp.zeros_like(sst_ref)
        sts_ref[...] = jnp.zeros_like(sts_ref)

    sst_ref[...] += upd_st
    sts_ref[...] += upd_ts


def _k2(g1, g2, ear, wc, bm1, be):
    e = g1.shape[0]
    nb = e // be
    return pl.pallas_call(
        functools.partial(_k2_body, be=be),
        grid=(nb,),
        in_specs=[
            pl.BlockSpec((be, D), lambda i: (i, 0)),
            pl.BlockSpec((be, D), lambda i: (i, 0)),
            pl.BlockSpec((be, 4), lambda i: (i, 0)),
            pl.BlockSpec((4, D), lambda i: (0, 0)),
            pl.BlockSpec((1, D), lambda i: (0, 0)),
        ],
        out_specs=[
            pl.BlockSpec((be, D), lambda i: (i, 0)),
            pl.BlockSpec((2, D), lambda i: (0, 0)),
            pl.BlockSpec((2, D), lambda i: (0, 0)),
        ],
        out_shape=[
            jax.ShapeDtypeStruct((e, D), jnp.int32),
            jax.ShapeDtypeStruct((2, D), jnp.float32),
            jax.ShapeDtypeStruct((2, D), jnp.float32),
        ],
    )(g1, g2, ear, wc, bm1)


def _scale_shift(stats, g, b, count):
    m = stats[0:1, :] / count
    v = stats[1:2, :] / count - m * m
    scale = g * lax.rsqrt(v + EPS)
    shift = b - m * scale
    return scale, shift


def _k3_body(y1p_ref, s1st_ref, s1ts_ref, gm1_ref, bem1_ref,
             wm2_ref, bm2_ref, y2p_ref, sst_ref, sts_ref, *, count):
    i = pl.program_id(0)
    g = gm1_ref[...]
    b = bem1_ref[...]
    sc_st, sh_st = _scale_shift(s1st_ref[...], g, b, count)
    sc_ts, sh_ts = _scale_shift(s1ts_ref[...], g, b, count)
    y1p = y1p_ref[...]
    z_st = jnp.maximum(_unpack_hi(y1p) * sc_st + sh_st, 0.0)
    z_ts = jnp.maximum(_unpack_lo(y1p) * sc_ts + sh_ts, 0.0)
    w = wm2_ref[...].astype(jnp.bfloat16)
    y2st = jnp.dot(z_st.astype(jnp.bfloat16), w,
                   preferred_element_type=jnp.float32) + bm2_ref[...]
    y2ts = jnp.dot(z_ts.astype(jnp.bfloat16), w,
                   preferred_element_type=jnp.float32) + bm2_ref[...]
    y2p_ref[...] = _pack2(y2st, y2ts)
    upd_st = jnp.concatenate(
        [jnp.sum(y2st, 0, keepdims=True), jnp.sum(y2st * y2st, 0, keepdims=True)], 0)
    upd_ts = jnp.concatenate(
        [jnp.sum(y2ts, 0, keepdims=True), jnp.sum(y2ts * y2ts, 0, keepdims=True)], 0)

    @pl.when(i == 0)
    def _():
        sst_ref[...] = jnp.zeros_like(sst_ref)
        sts_ref[...] = jnp.zeros_like(sts_ref)

    sst_ref[...] += upd_st
    sts_ref[...] += upd_ts


def _k3(y1p, s1st, s1ts, gm1, bem1, wm2, bm2, be):
    e = y1p.shape[0]
    nb = e // be
    blk = lambda i: (i, 0)
    fix = lambda i: (0, 0)
    return pl.pallas_call(
        functools.partial(_k3_body, count=float(e)),
        grid=(nb,),
        in_specs=[
            pl.BlockSpec((be, D), blk),
            pl.BlockSpec((2, D), fix),
            pl.BlockSpec((2, D), fix),
            pl.BlockSpec((1, D), fix),
            pl.BlockSpec((1, D), fix),
            pl.BlockSpec((D, D), fix),
            pl.BlockSpec((1, D), fix),
        ],
        out_specs=[
            pl.BlockSpec((be, D), blk),
            pl.BlockSpec((2, D), fix),
            pl.BlockSpec((2, D), fix),
        ],
        out_shape=[
            jax.ShapeDtypeStruct((e, D), jnp.int32),
            jax.ShapeDtypeStruct((2, D), jnp.float32),
            jax.ShapeDtypeStruct((2, D), jnp.float32),
        ],
    )(y1p, s1st, s1ts, gm1, bem1, wm2, bm2)


def _k3b_body(y2p_ref, s2st_ref, s2ts_ref, gm2_ref, bem2_ref,
              mst_ref, mts_ref, *, count):
    g = gm2_ref[...]
    b = bem2_ref[...]
    sc_st, sh_st = _scale_shift(s2st_ref[...], g, b, count)
    sc_ts, sh_ts = _scale_shift(s2ts_ref[...], g, b, count)
    y2p = y2p_ref[...]
    mst_ref[...] = jnp.maximum(_unpack_hi(y2p) * sc_st + sh_st, 0.0)
    mts_ref[...] = jnp.maximum(_unpack_lo(y2p) * sc_ts + sh_ts, 0.0)


def _k3b(y2p, s2st, s2ts, gm2, bem2, be):
    e = y2p.shape[0]
    nb = e // be
    blk = lambda i: (i, 0)
    fix = lambda i: (0, 0)
    return pl.pallas_call(
        functools.partial(_k3b_body, count=float(e)),
        grid=(nb,),
        in_specs=[
            pl.BlockSpec((be, D), blk),
            pl.BlockSpec((2, D), fix),
            pl.BlockSpec((2, D), fix),
            pl.BlockSpec((1, D), fix),
            pl.BlockSpec((1, D), fix),
        ],
        out_specs=[
            pl.BlockSpec((be, D), blk),
            pl.BlockSpec((be, D), blk),
        ],
        out_shape=[
            jax.ShapeDtypeStruct((e, D), jnp.float32),
            jax.ShapeDtypeStruct((e, D), jnp.float32),
        ],
    )(y2p, s2st, s2ts, gm2, bem2)


def _sc_scatter(mst, mts, src, zeros_n):
    e = src.shape[0]
    n = zeros_n.shape[0]
    info = plsc.get_sparse_core_info()
    nc, ns = info.num_cores, info.num_subcores
    ept = e // ns
    chunk = 160
    nch = ept // chunk
    npt = (n // ns) // 8 * 8
    tail = n - ns * npt
    mesh = plsc.VectorSubcoreMesh(core_axis_name="c", subcore_axis_name="s")

    @functools.partial(
        pl.kernel,
        mesh=mesh,
        out_type=[jax.ShapeDtypeStruct((n, D), jnp.float32),
                  jax.ShapeDtypeStruct((n, D), jnp.float32)],
        scratch_types=[
            pltpu.VMEM((chunk,), jnp.int32),
            pltpu.VMEM((chunk,), jnp.int32),
            pltpu.VMEM((chunk, D), jnp.float32),
            pltpu.VMEM((chunk, D), jnp.float32),
            pltpu.VMEM_SHARED((n, D), jnp.float32),
            pltpu.SemaphoreType.DMA,
            pltpu.SemaphoreType.DMA,
        ],
    )
    def k(mst_hbm, mts_hbm, src_hbm, zeros_hbm, ast_hbm, ats_hbm,
          ia, ib, va, vb, aggr_sh, sa, sb):
        c = lax.axis_index("c")
        s = lax.axis_index("s")
        rs = s * npt
        pltpu.sync_copy(zeros_hbm.at[pl.ds(rs, npt)],
                        aggr_sh.at[pl.ds(rs, npt)])

        @pl.when(s == 0)
        def _():
            pltpu.sync_copy(zeros_hbm.at[pl.ds(ns * npt, tail)],
                            aggr_sh.at[pl.ds(ns * npt, tail)])

        plsc.subcore_barrier()

        base = s * ept

        def load(ci, iv, vv, sem):
            o = base + ci * chunk
            pltpu.sync_copy(src_hbm.at[pl.ds(o, chunk)], iv)

            @pl.when(c == 0)
            def _():
                pltpu.async_copy(mst_hbm.at[pl.ds(o, chunk)], vv, sem)

            @pl.when(c != 0)
            def _():
                pltpu.async_copy(mts_hbm.at[pl.ds(o, chunk)], vv, sem)

        def wait_v(vv, sem):
            pltpu.make_async_copy(mst_hbm.at[pl.ds(0, chunk)], vv, sem).wait()

        load(0, ia, va, sa)

        def body(j, carry):
            ca = 2 * j
            cb = ca + 1
            wait_v(va, sa)
            load(cb, ib, vb, sb)
            pltpu.sync_copy(va, aggr_sh.at[ia], add=True)
            wait_v(vb, sb)

            @pl.when(cb + 1 < nch)
            def _():
                load(cb + 1, ia, va, sa)

            pltpu.sync_copy(vb, aggr_sh.at[ib], add=True)
            return carry

        lax.fori_loop(0, nch // 2, body, 0)

        if nch % 2 == 1:
            wait_v(va, sa)
            pltpu.sync_copy(va, aggr_sh.at[ia], add=True)

        plsc.subcore_barrier()

        @pl.when(c == 0)
        def _():
            pltpu.sync_copy(aggr_sh.at[pl.ds(rs, npt)],
                            ast_hbm.at[pl.ds(rs, npt)])

            @pl.when(s == 0)
            def _():
                pltpu.sync_copy(aggr_sh.at[pl.ds(ns * npt, tail)],
                                ast_hbm.at[pl.ds(ns * npt, tail)])

        @pl.when(c != 0)
        def _():
            pltpu.sync_copy(aggr_sh.at[pl.ds(rs, npt)],
                            ats_hbm.at[pl.ds(rs, npt)])

            @pl.when(s == 0)
            def _():
                pltpu.sync_copy(aggr_sh.at[pl.ds(ns * npt, tail)],
                                ats_hbm.at[pl.ds(ns * npt, tail)])

    return k(mst, mts, src, zeros_n)


def _k5_body(h_ref, ast_ref, ats_ref, wu1_ref, bu1_ref, gu1_ref, beu1_ref,
             wu2_ref, bu2_ref, gu2_ref, beu2_ref, out_ref):
    h = h_ref[...]
    wu1 = wu1_ref[...]
    u = (jnp.dot(h, wu1[:D], preferred_element_type=jnp.float32)
         + jnp.dot(ast_ref[...], ALPHA * wu1[D:2 * D],
                   preferred_element_type=jnp.float32)
         + jnp.dot(ats_ref[...], (1.0 - ALPHA) * wu1[2 * D:],
                   preferred_element_type=jnp.float32)
         + bu1_ref[...])
    m = jnp.mean(u, axis=0, keepdims=True)
    v = jnp.mean((u - m) * (u - m), axis=0, keepdims=True)
    z = jnp.maximum(gu1_ref[...] * (u - m) * lax.rsqrt(v + EPS)
                    + beu1_ref[...], 0.0)
    u2 = jnp.dot(z, wu2_ref[...], preferred_element_type=jnp.float32) \
        + bu2_ref[...]
    m2 = jnp.mean(u2, axis=0, keepdims=True)
    v2 = jnp.mean((u2 - m2) * (u2 - m2), axis=0, keepdims=True)
    out_ref[...] = jnp.maximum(
        gu2_ref[...] * (u2 - m2) * lax.rsqrt(v2 + EPS) + beu2_ref[...], 0.0)


def _k5(h, ast, ats, wu1, bu1, gu1, beu1, wu2, bu2, gu2, beu2):
    n = h.shape[0]
    return pl.pallas_call(
        _k5_body,
        out_shape=jax.ShapeDtypeStruct((n, D), jnp.float32),
    )(h, ast, ats, wu1, bu1, gu1, beu1, wu2, bu2, gu2, beu2)


_BE = 8000


def kernel(h, edge_index, edge_attr, Wm1, bm1, gm1, bem1, Wm2, bm2, gm2, bem2,
           Wu1, bu1, gu1, beu1, Wu2, bu2, gu2, beu2):
    n = h.shape[0]
    e = edge_index.shape[1]
    src = edge_index[0]
    dst = edge_index[1]
    wq = jnp.concatenate([Wm1[:D], Wm1[D:2 * D]], axis=1)
    wc = Wm1[2 * D:]
    ear = edge_attr
    row = lambda x: x.reshape(1, D)

    q = _k0(h, wq)
    g1, g2 = _sc_gather(q, src, dst)
    y1p, s1st, s1ts = _k2(g1, g2, ear, wc, row(bm1), _BE)
    y2p, s2st, s2ts = _k3(y1p, s1st, s1ts, row(gm1), row(bem1),
                          Wm2, row(bm2), _BE)
    mst, mts = _k3b(y2p, s2st, s2ts, row(gm2), row(bem2), _BE)
    zeros_n = jnp.zeros((n, D), jnp.float32)
    ast, ats = _sc_scatter(mst, mts, src, zeros_n)
    out = _k5(h, ast, ats, Wu1, row(bu1), row(gu1), row(beu1),
              Wu2, row(bu2), row(gu2), row(beu2))
    return out


# speedup vs baseline: 1.1921x; 1.0027x over previous
import functools

import jax
import jax.numpy as jnp
from jax import lax
from jax.experimental import pallas as pl
from jax.experimental.pallas import tpu as pltpu
from jax.experimental.pallas import tpu_sc as plsc

D = 128
EPS = 1e-5
ALPHA = 0.5


_HI_MASK = -65536


def _bf16_bits(x):
    xi = lax.bitcast_convert_type(x, jnp.int32)
    return (xi + 0x7FFF + ((xi >> 16) & 1)) & _HI_MASK


def _pack2(a, b):
    return _bf16_bits(a) | ((_bf16_bits(b) >> 16) & 0xFFFF)


def _unpack_hi(w):
    return lax.bitcast_convert_type(w & _HI_MASK, jnp.float32)


def _unpack_lo(w):
    return lax.bitcast_convert_type(w << 16, jnp.float32)


def _k0_body(h_ref, wq_ref, q_ref):
    q = jnp.dot(h_ref[...], wq_ref[...], preferred_element_type=jnp.float32)
    q_ref[...] = _pack2(q[:, :D], q[:, D:])


def _k0(h, wq):
    n = h.shape[0]
    return pl.pallas_call(
        _k0_body,
        out_shape=jax.ShapeDtypeStruct((n, D), jnp.int32),
    )(h, wq)


def _sc_gather(q, src, dst):
    e = src.shape[0]
    info = plsc.get_sparse_core_info()
    nc, ns = info.num_cores, info.num_subcores
    nw = nc * ns
    epw = e // nw
    chunk = 200
    nch = epw // chunk
    mesh = plsc.VectorSubcoreMesh(core_axis_name="c", subcore_axis_name="s")

    @functools.partial(
        pl.kernel,
        mesh=mesh,
        out_type=[jax.ShapeDtypeStruct((e, D), jnp.int32),
                  jax.ShapeDtypeStruct((e, D), jnp.int32)],
        scratch_types=[
            pltpu.VMEM((chunk,), jnp.int32),
            pltpu.VMEM((chunk,), jnp.int32),
            pltpu.VMEM((chunk,), jnp.int32),
            pltpu.VMEM((chunk,), jnp.int32),
            pltpu.VMEM((chunk, D), jnp.int32),
            pltpu.VMEM((chunk, D), jnp.int32),
            pltpu.VMEM((chunk, D), jnp.int32),
            pltpu.VMEM((chunk, D), jnp.int32),
            pltpu.SemaphoreType.DMA,
            pltpu.SemaphoreType.DMA,
            pltpu.SemaphoreType.DMA,
            pltpu.SemaphoreType.DMA,
        ],
    )
    def k(q_hbm, src_hbm, dst_hbm, g1_hbm, g2_hbm,
          i1a, i2a, i1b, i2b, r1a, r2a, r1b, r2b,
          s1a, s2a, s1b, s2b):
        wid = lax.axis_index("s") * nc + lax.axis_index("c")
        base = wid * epw

        def load_gather(c, i1, i2, r1, r2, s1, s2):
            o = base + c * chunk
            pltpu.sync_copy(src_hbm.at[pl.ds(o, chunk)], i1)
            pltpu.sync_copy(dst_hbm.at[pl.ds(o, chunk)], i2)
            pltpu.async_copy(q_hbm.at[i1], r1, s1)
            pltpu.async_copy(q_hbm.at[i2], r2, s2)

        def wait_g(r1, r2, s1, s2):
            pltpu.make_async_copy(g1_hbm.at[pl.ds(0, chunk)], r1, s1).wait()
            pltpu.make_async_copy(g1_hbm.at[pl.ds(0, chunk)], r2, s2).wait()

        def write(c, r1, r2):
            o = base + c * chunk
            pltpu.sync_copy(r1, g1_hbm.at[pl.ds(o, chunk)])
            pltpu.sync_copy(r2, g2_hbm.at[pl.ds(o, chunk)])

        load_gather(0, i1a, i2a, r1a, r2a, s1a, s2a)

        def body(j, carry):
            ca = 2 * j
            cb = ca + 1
            wait_g(r1a, r2a, s1a, s2a)
            load_gather(cb, i1b, i2b, r1b, r2b, s1b, s2b)
            write(ca, r1a, r2a)
            wait_g(r1b, r2b, s1b, s2b)

            @pl.when(cb + 1 < nch)
            def _():
                load_gather(cb + 1, i1a, i2a, r1a, r2a, s1a, s2a)

            write(cb, r1b, r2b)
            return carry

        lax.fori_loop(0, nch // 2, body, 0)

    return k(q, src, dst)


def _k2_body(g1_ref, g2_ref, ear_ref, wc_ref, bm1_ref,
             y1p_ref, sst_ref, sts_ref, *, be):
    i = pl.program_id(0)
    ew = jnp.dot(ear_ref[...], wc_ref[...],
                 preferred_element_type=jnp.float32) + bm1_ref[...]
    g1 = g1_ref[...]
    g2 = g2_ref[...]
    yst = _unpack_hi(g1) + _unpack_lo(g2) + ew
    yts = _unpack_hi(g2) + _unpack_lo(g1) + ew
    y1p_ref[...] = _pack2(yst, yts)
    upd_st = jnp.concatenate(
        [jnp.sum(yst, 0, keepdims=True), jnp.sum(yst * yst, 0, keepdims=True)], 0)
    upd_ts = jnp.concatenate(
        [jnp.sum(yts, 0, keepdims=True), jnp.sum(yts * yts, 0, keepdims=True)], 0)

    @pl.when(i == 0)
    def _():
        sst_ref[...] = jnp.zeros_like(sst_ref)
        sts_ref[...] = jnp.zeros_like(sts_ref)

    sst_ref[...] += upd_st
    sts_ref[...] += upd_ts


def _k2(g1, g2, ear, wc, bm1, be):
    e = g1.shape[0]
    nb = e // be
    return pl.pallas_call(
        functools.partial(_k2_body, be=be),
        grid=(nb,),
        in_specs=[
            pl.BlockSpec((be, D), lambda i: (i, 0)),
            pl.BlockSpec((be, D), lambda i: (i, 0)),
            pl.BlockSpec((be, 4), lambda i: (i, 0)),
            pl.BlockSpec((4, D), lambda i: (0, 0)),
            pl.BlockSpec((1, D), lambda i: (0, 0)),
        ],
        out_specs=[
            pl.BlockSpec((be, D), lambda i: (i, 0)),
            pl.BlockSpec((2, D), lambda i: (0, 0)),
            pl.BlockSpec((2, D), lambda i: (0, 0)),
        ],
        out_shape=[
            jax.ShapeDtypeStruct((e, D), jnp.int32),
            jax.ShapeDtypeStruct((2, D), jnp.float32),
            jax.ShapeDtypeStruct((2, D), jnp.float32),
        ],
    )(g1, g2, ear, wc, bm1)


def _scale_shift(stats, g, b, count):
    m = stats[0:1, :] / count
    v = stats[1:2, :] / count - m * m
    scale = g * lax.rsqrt(v + EPS)
    shift = b - m * scale
    return scale, shift


def _k3_body(y1p_ref, s1st_ref, s1ts_ref, gm1_ref, bem1_ref,
             wm2_ref, bm2_ref, y2p_ref, sst_ref, sts_ref, *, count):
    i = pl.program_id(0)
    g = gm1_ref[...]
    b = bem1_ref[...]
    sc_st, sh_st = _scale_shift(s1st_ref[...], g, b, count)
    sc_ts, sh_ts = _scale_shift(s1ts_ref[...], g, b, count)
    y1p = y1p_ref[...]
    z_st = jnp.maximum(_unpack_hi(y1p) * sc_st + sh_st, 0.0)
    z_ts = jnp.maximum(_unpack_lo(y1p) * sc_ts + sh_ts, 0.0)
    w = wm2_ref[...].astype(jnp.bfloat16)
    y2st = jnp.dot(z_st.astype(jnp.bfloat16), w,
                   preferred_element_type=jnp.float32) + bm2_ref[...]
    y2ts = jnp.dot(z_ts.astype(jnp.bfloat16), w,
                   preferred_element_type=jnp.float32) + bm2_ref[...]
    y2p_ref[...] = _pack2(y2st, y2ts)
    upd_st = jnp.concatenate(
        [jnp.sum(y2st, 0, keepdims=True), jnp.sum(y2st * y2st, 0, keepdims=True)], 0)
    upd_ts = jnp.concatenate(
        [jnp.sum(y2ts, 0, keepdims=True), jnp.sum(y2ts * y2ts, 0, keepdims=True)], 0)

    @pl.when(i == 0)
    def _():
        sst_ref[...] = jnp.zeros_like(sst_ref)
        sts_ref[...] = jnp.zeros_like(sts_ref)

    sst_ref[...] += upd_st
    sts_ref[...] += upd_ts


def _k3(y1p, s1st, s1ts, gm1, bem1, wm2, bm2, be):
    e = y1p.shape[0]
    nb = e // be
    blk = lambda i: (i, 0)
    fix = lambda i: (0, 0)
    return pl.pallas_call(
        functools.partial(_k3_body, count=float(e)),
        grid=(nb,),
        in_specs=[
            pl.BlockSpec((be, D), blk),
            pl.BlockSpec((2, D), fix),
            pl.BlockSpec((2, D), fix),
            pl.BlockSpec((1, D), fix),
            pl.BlockSpec((1, D), fix),
            pl.BlockSpec((D, D), fix),
            pl.BlockSpec((1, D), fix),
        ],
        out_specs=[
            pl.BlockSpec((be, D), blk),
            pl.BlockSpec((2, D), fix),
            pl.BlockSpec((2, D), fix),
        ],
        out_shape=[
            jax.ShapeDtypeStruct((e, D), jnp.int32),
            jax.ShapeDtypeStruct((2, D), jnp.float32),
            jax.ShapeDtypeStruct((2, D), jnp.float32),
        ],
    )(y1p, s1st, s1ts, gm1, bem1, wm2, bm2)


def _k3b_body(y2p_ref, s2st_ref, s2ts_ref, gm2_ref, bem2_ref,
              mst_ref, mts_ref, *, count):
    g = gm2_ref[...]
    b = bem2_ref[...]
    sc_st, sh_st = _scale_shift(s2st_ref[...], g, b, count)
    sc_ts, sh_ts = _scale_shift(s2ts_ref[...], g, b, count)
    y2p = y2p_ref[...]
    mst_ref[...] = jnp.maximum(_unpack_hi(y2p) * sc_st + sh_st, 0.0)
    mts_ref[...] = jnp.maximum(_unpack_lo(y2p) * sc_ts + sh_ts, 0.0)


def _k3b(y2p, s2st, s2ts, gm2, bem2, be):
    e = y2p.shape[0]
    nb = e // be
    blk = lambda i: (i, 0)
    fix = lambda i: (0, 0)
    return pl.pallas_call(
        functools.partial(_k3b_body, count=float(e)),
        grid=(nb,),
        in_specs=[
            pl.BlockSpec((be, D), blk),
            pl.BlockSpec((2, D), fix),
            pl.BlockSpec((2, D), fix),
            pl.BlockSpec((1, D), fix),
            pl.BlockSpec((1, D), fix),
        ],
        out_specs=[
            pl.BlockSpec((be, D), blk),
            pl.BlockSpec((be, D), blk),
        ],
        out_shape=[
            jax.ShapeDtypeStruct((e, D), jnp.float32),
            jax.ShapeDtypeStruct((e, D), jnp.float32),
        ],
    )(y2p, s2st, s2ts, gm2, bem2)


def _sc_scatter(mst, mts, src, zeros_n):
    e = src.shape[0]
    n = zeros_n.shape[0]
    info = plsc.get_sparse_core_info()
    nc, ns = info.num_cores, info.num_subcores
    ept = e // ns
    chunk = 160
    nch = ept // chunk
    npt = (n // ns) // 8 * 8
    tail = n - ns * npt
    mesh = plsc.VectorSubcoreMesh(core_axis_name="c", subcore_axis_name="s")

    @functools.partial(
        pl.kernel,
        mesh=mesh,
        out_type=[jax.ShapeDtypeStruct((n, D), jnp.float32),
                  jax.ShapeDtypeStruct((n, D), jnp.float32)],
        scratch_types=[
            pltpu.VMEM((chunk,), jnp.int32),
            pltpu.VMEM((chunk,), jnp.int32),
            pltpu.VMEM((chunk, D), jnp.float32),
            pltpu.VMEM((chunk, D), jnp.float32),
            pltpu.VMEM_SHARED((n, D), jnp.float32),
            pltpu.SemaphoreType.DMA,
            pltpu.SemaphoreType.DMA,
        ],
    )
    def k(mst_hbm, mts_hbm, src_hbm, zeros_hbm, ast_hbm, ats_hbm,
          ia, ib, va, vb, aggr_sh, sa, sb):
        c = lax.axis_index("c")
        s = lax.axis_index("s")
        rs = s * npt
        pltpu.sync_copy(zeros_hbm.at[pl.ds(rs, npt)],
                        aggr_sh.at[pl.ds(rs, npt)])

        @pl.when(s == 0)
        def _():
            pltpu.sync_copy(zeros_hbm.at[pl.ds(ns * npt, tail)],
                            aggr_sh.at[pl.ds(ns * npt, tail)])

        plsc.subcore_barrier()

        base = s * ept

        def load(ci, iv, vv, sem):
            o = base + ci * chunk
            pltpu.sync_copy(src_hbm.at[pl.ds(o, chunk)], iv)

            @pl.when(c == 0)
            def _():
                pltpu.async_copy(mst_hbm.at[pl.ds(o, chunk)], vv, sem)

            @pl.when(c != 0)
            def _():
                pltpu.async_copy(mts_hbm.at[pl.ds(o, chunk)], vv, sem)

        def wait_v(vv, sem):
            pltpu.make_async_copy(mst_hbm.at[pl.ds(0, chunk)], vv, sem).wait()

        load(0, ia, va, sa)

        def body(j, carry):
            ca = 2 * j
            cb = ca + 1
            wait_v(va, sa)
            load(cb, ib, vb, sb)
            pltpu.sync_copy(va, aggr_sh.at[ia], add=True)
            wait_v(vb, sb)

            @pl.when(cb + 1 < nch)
            def _():
                load(cb + 1, ia, va, sa)

            pltpu.sync_copy(vb, aggr_sh.at[ib], add=True)
            return carry

        lax.fori_loop(0, nch // 2, body, 0)

        if nch % 2 == 1:
            wait_v(va, sa)
            pltpu.sync_copy(va, aggr_sh.at[ia], add=True)

        plsc.subcore_barrier()

        @pl.when(c == 0)
        def _():
            pltpu.sync_copy(aggr_sh.at[pl.ds(rs, npt)],
                            ast_hbm.at[pl.ds(rs, npt)])

            @pl.when(s == 0)
            def _():
                pltpu.sync_copy(aggr_sh.at[pl.ds(ns * npt, tail)],
                                ast_hbm.at[pl.ds(ns * npt, tail)])

        @pl.when(c != 0)
        def _():
            pltpu.sync_copy(aggr_sh.at[pl.ds(rs, npt)],
                            ats_hbm.at[pl.ds(rs, npt)])

            @pl.when(s == 0)
            def _():
                pltpu.sync_copy(aggr_sh.at[pl.ds(ns * npt, tail)],
                                ats_hbm.at[pl.ds(ns * npt, tail)])

    return k(mst, mts, src, zeros_n)


def _k5_body(h_ref, ast_ref, ats_ref, wu1_ref, bu1_ref, gu1_ref, beu1_ref,
             wu2_ref, bu2_ref, gu2_ref, beu2_ref, out_ref):
    h = h_ref[...]
    wu1 = wu1_ref[...]
    u = (jnp.dot(h, wu1[:D], preferred_element_type=jnp.float32)
         + jnp.dot(ast_ref[...], ALPHA * wu1[D:2 * D],
                   preferred_element_type=jnp.float32)
         + jnp.dot(ats_ref[...], (1.0 - ALPHA) * wu1[2 * D:],
                   preferred_element_type=jnp.float32)
         + bu1_ref[...])
    m = jnp.mean(u, axis=0, keepdims=True)
    v = jnp.mean((u - m) * (u - m), axis=0, keepdims=True)
    z = jnp.maximum(gu1_ref[...] * (u - m) * lax.rsqrt(v + EPS)
                    + beu1_ref[...], 0.0)
    u2 = jnp.dot(z, wu2_ref[...], preferred_element_type=jnp.float32) \
        + bu2_ref[...]
    m2 = jnp.mean(u2, axis=0, keepdims=True)
    v2 = jnp.mean((u2 - m2) * (u2 - m2), axis=0, keepdims=True)
    out_ref[...] = jnp.maximum(
        gu2_ref[...] * (u2 - m2) * lax.rsqrt(v2 + EPS) + beu2_ref[...], 0.0)


def _k5(h, ast, ats, wu1, bu1, gu1, beu1, wu2, bu2, gu2, beu2):
    n = h.shape[0]
    return pl.pallas_call(
        _k5_body,
        out_shape=jax.ShapeDtypeStruct((n, D), jnp.float32),
    )(h, ast, ats, wu1, bu1, gu1, beu1, wu2, bu2, gu2, beu2)


_BE = 10000


def kernel(h, edge_index, edge_attr, Wm1, bm1, gm1, bem1, Wm2, bm2, gm2, bem2,
           Wu1, bu1, gu1, beu1, Wu2, bu2, gu2, beu2):
    n = h.shape[0]
    e = edge_index.shape[1]
    src = edge_index[0]
    dst = edge_index[1]
    wq = jnp.concatenate([Wm1[:D], Wm1[D:2 * D]], axis=1)
    wc = Wm1[2 * D:]
    ear = edge_attr
    row = lambda x: x.reshape(1, D)

    q = _k0(h, wq)
    g1, g2 = _sc_gather(q, src, dst)
    y1p, s1st, s1ts = _k2(g1, g2, ear, wc, row(bm1), _BE)
    y2p, s2st, s2ts = _k3(y1p, s1st, s1ts, row(gm1), row(bem1),
                          Wm2, row(bm2), _BE)
    mst, mts = _k3b(y2p, s2st, s2ts, row(gm2), row(bem2), _BE)
    zeros_n = jnp.zeros((n, D), jnp.float32)
    ast, ats = _sc_scatter(mst, mts, src, zeros_n)
    out = _k5(h, ast, ats, Wu1, row(bu1), row(gu1), row(beu1),
              Wu2, row(bu2), row(gu2), row(beu2))
    return out
